# Initial kernel scaffold; baseline (speedup 1.0000x reference)
import jax
import jax.numpy as jnp
from jax import lax
from jax.experimental import pallas as pl
from jax.experimental.pallas import tpu as pltpu
from jax.experimental.pallas import tpu_sc as plsc

N_NODES = 10000
N_EDGES = 320000
F = 128

NUM_CORES = 2
NUM_SUBCORES = 16
NW = NUM_CORES * NUM_SUBCORES
K = 128
CH_PER_W = 79
TOT_CH = NW * CH_PER_W
PAD_E = TOT_CH * K - N_EDGES
ROWS_PER_TILE = N_NODES // NUM_SUBCORES


def _sc_body(x_hbm, src_hbm, dst_hbm, out_hbm, accum, src_idx, dst_idx, rows, sem):
    cid = lax.axis_index("c")
    sid = lax.axis_index("s")
    wid = cid * NUM_SUBCORES + sid

    zeros16 = jnp.zeros((16,), jnp.float32)

    def zero_row(r, carry):
        for c in range(F // 16):
            rows[r, pl.ds(c * 16, 16)] = zeros16
        return carry

    lax.fori_loop(0, K, zero_row, 0)
    base = sid * ROWS_PER_TILE
    for off, length in ((0, 128), (128, 128), (256, 128), (384, 128), (512, 113)):
        pltpu.sync_copy(rows.at[pl.ds(0, length)],
                        accum.at[pl.ds(base + off, length)])
    plsc.subcore_barrier()

    pltpu.sync_copy(src_hbm.at[pl.ds(wid * CH_PER_W, CH_PER_W)], src_idx)
    pltpu.sync_copy(dst_hbm.at[pl.ds(wid * CH_PER_W, CH_PER_W)], dst_idx)

    def edge_chunk(j, carry):
        pltpu.async_copy(x_hbm.at[src_idx.at[j]], rows, sem).wait()
        pltpu.sync_copy(rows, accum.at[dst_idx.at[j]], add=True)
        return carry

    lax.fori_loop(0, CH_PER_W, edge_chunk, 0)
    plsc.subcore_barrier()

    pltpu.sync_copy(accum.at[pl.ds(base, ROWS_PER_TILE)],
                    out_hbm.at[cid, pl.ds(base, ROWS_PER_TILE)])


_sc_aggregate = pl.kernel(
    _sc_body,
    out_type=jax.ShapeDtypeStruct((NUM_CORES, N_NODES, F), jnp.float32),
    mesh=plsc.VectorSubcoreMesh(core_axis_name="c", subcore_axis_name="s"),
    scratch_types=[
        pltpu.VMEM_SHARED((N_NODES + 1, F), jnp.float32),
        pltpu.VMEM((CH_PER_W, K), jnp.int32),
        pltpu.VMEM((CH_PER_W, K), jnp.int32),
        pltpu.VMEM((K, F), jnp.float32),
        pltpu.SemaphoreType.DMA,
    ],
)


def _tc_matmul_body(p_ref, w_ref, b_ref, o_ref):
    s = p_ref[0] + p_ref[1]
    o_ref[...] = jnp.dot(s, w_ref[...],
                         preferred_element_type=jnp.float32) + b_ref[...]


BM = 1000

_tc_matmul = pl.pallas_call(
    _tc_matmul_body,
    grid=(N_NODES // BM,),
    in_specs=[
        pl.BlockSpec((NUM_CORES, BM, F), lambda i: (0, i, 0)),
        pl.BlockSpec((F, F), lambda i: (0, 0)),
        pl.BlockSpec((1, F), lambda i: (0, 0)),
    ],
    out_specs=pl.BlockSpec((BM, F), lambda i: (i, 0)),
    out_shape=jax.ShapeDtypeStruct((N_NODES, F), jnp.float32),
)


@jax.jit
def kernel(x, edge_index, weight, bias):
    src = edge_index[0]
    dst = edge_index[1]
    src_p = jnp.concatenate(
        [src, jnp.zeros((PAD_E,), jnp.int32)]).reshape(TOT_CH, K)
    dst_p = jnp.concatenate(
        [dst, jnp.full((PAD_E,), N_NODES, jnp.int32)]).reshape(TOT_CH, K)
    partial = _sc_aggregate(x, src_p, dst_p)
    return _tc_matmul(partial, weight, bias.reshape(1, F))


# speedup vs baseline: 3.0293x; 3.0293x over previous
import jax
import jax.numpy as jnp
from jax import lax
from jax.experimental import pallas as pl
from jax.experimental.pallas import tpu as pltpu
from jax.experimental.pallas import tpu_sc as plsc

N_NODES = 10000
N_EDGES = 320000
F = 128

NUM_CORES = 2
NUM_SUBCORES = 16
NW = NUM_CORES * NUM_SUBCORES
K = 128
CH_PER_W = 80
TOT_CH = NW * CH_PER_W
PAD_E = TOT_CH * K - N_EDGES
STRIP = 632
ACC_ROWS = STRIP * NUM_SUBCORES


def _sc_body(x_hbm, src_hbm, dst_hbm, out_hbm, accum, src_idx, dst_idx, rows, sem):
    cid = lax.axis_index("c")
    sid = lax.axis_index("s")
    wid = cid * NUM_SUBCORES + sid

    zeros16 = jnp.zeros((16,), jnp.float32)

    def zero_row(r, carry):
        for c in range(F // 16):
            rows[r, pl.ds(c * 16, 16)] = zeros16
        return carry

    lax.fori_loop(0, K, zero_row, 0)
    base = sid * STRIP
    for off, length in ((0, 128), (128, 128), (256, 128), (384, 128), (512, 120)):
        pltpu.sync_copy(rows.at[pl.ds(0, length)],
                        accum.at[pl.ds(base + off, length)])
    plsc.subcore_barrier()

    pltpu.sync_copy(src_hbm.at[pl.ds(wid * CH_PER_W, CH_PER_W)], src_idx)
    pltpu.sync_copy(dst_hbm.at[pl.ds(wid * CH_PER_W, CH_PER_W)], dst_idx)

    def edge_chunk(j, carry):
        pltpu.async_copy(x_hbm.at[src_idx.at[j]], rows, sem).wait()
        pltpu.sync_copy(rows, accum.at[dst_idx.at[j]], add=True)
        return carry

    lax.fori_loop(0, CH_PER_W, edge_chunk, 0)
    plsc.subcore_barrier()

    pltpu.sync_copy(accum.at[pl.ds(base, STRIP)],
                    out_hbm.at[cid, pl.ds(base, STRIP)])


_sc_aggregate = pl.kernel(
    _sc_body,
    out_type=jax.ShapeDtypeStruct((NUM_CORES, ACC_ROWS, F), jnp.float32),
    mesh=plsc.VectorSubcoreMesh(core_axis_name="c", subcore_axis_name="s"),
    scratch_types=[
        pltpu.VMEM_SHARED((ACC_ROWS, F), jnp.float32),
        pltpu.VMEM((CH_PER_W, K), jnp.int32),
        pltpu.VMEM((CH_PER_W, K), jnp.int32),
        pltpu.VMEM((K, F), jnp.float32),
        pltpu.SemaphoreType.DMA,
    ],
)


def _tc_matmul_body(p_ref, w_ref, b_ref, o_ref):
    s = p_ref[0] + p_ref[1]
    o_ref[...] = jnp.dot(s, w_ref[...],
                         preferred_element_type=jnp.float32) + b_ref[...]


BM = 1000

_tc_matmul = pl.pallas_call(
    _tc_matmul_body,
    grid=(N_NODES // BM,),
    in_specs=[
        pl.BlockSpec((NUM_CORES, BM, F), lambda i: (0, i, 0)),
        pl.BlockSpec((F, F), lambda i: (0, 0)),
        pl.BlockSpec((1, F), lambda i: (0, 0)),
    ],
    out_specs=pl.BlockSpec((BM, F), lambda i: (i, 0)),
    out_shape=jax.ShapeDtypeStruct((N_NODES, F), jnp.float32),
)


@jax.jit
def kernel(x, edge_index, weight, bias):
    src = edge_index[0]
    dst = edge_index[1]
    src_p = jnp.concatenate(
        [src, jnp.zeros((PAD_E,), jnp.int32)]).reshape(TOT_CH, K)
    dst_p = jnp.concatenate(
        [dst, jnp.full((PAD_E,), N_NODES, jnp.int32)]).reshape(TOT_CH, K)
    partial = _sc_aggregate(x, src_p, dst_p)
    return _tc_matmul(partial, weight, bias.reshape(1, F))
